# baseline (speedup 1.0000x reference)
import jax
import jax.numpy as jnp
from jax.experimental import pallas as pl


def _copy_body(x_ref, o_ref):
    o_ref[...] = x_ref[...]


def kernel(method_soft, label, features):
    del label, features
    return pl.pallas_call(
        _copy_body,
        out_shape=jax.ShapeDtypeStruct(method_soft.shape, method_soft.dtype),
    )(method_soft)
